# baseline (speedup 1.0000x reference)
import functools

import jax
import jax.numpy as jnp
from jax import lax
from jax.experimental import pallas as pl
from jax.experimental.pallas import tpu as pltpu
from jax.experimental.pallas import tpu_sc as plsc

NW = 32
TABLE = 1_000_000
CHUNK = 62_496
SUB = 15_624
BN = 10_240


def _take_kernel(per_w):
    nb = per_w // BN
    mesh = plsc.VectorSubcoreMesh(core_axis_name="c", subcore_axis_name="s")

    @functools.partial(
        pl.kernel,
        mesh=mesh,
        out_type=jax.ShapeDtypeStruct((NW, per_w), jnp.float32),
        scratch_types=[
            pltpu.VMEM((BN,), jnp.int32),
            pltpu.VMEM((BN,), jnp.int32),
            pltpu.VMEM((BN,), jnp.float32),
            pltpu.VMEM((BN,), jnp.float32),
            pltpu.VMEM((SUB,), jnp.float32),
            pltpu.VMEM_SHARED((TABLE,), jnp.float32),
            pltpu.SemaphoreType.DMA,
            pltpu.SemaphoreType.DMA,
            pltpu.SemaphoreType.DMA,
        ],
    )
    def k(x_hbm, idx_hbm, out_hbm, idx_a, idx_b, out_a, out_b, bounce,
          table_sh, sem, sem_idx, sem_out):
        cid = lax.axis_index("c")
        sid = lax.axis_index("s")
        wid = sid * 2 + cid
        idx_bufs = (idx_a, idx_b)
        out_bufs = (out_a, out_b)

        def idx_load(blk, slot):
            return pltpu.make_async_copy(
                idx_hbm.at[wid, pl.ds(blk * BN, BN)], idx_bufs[slot],
                sem_idx)

        def out_store(blk, slot):
            return pltpu.make_async_copy(
                out_bufs[slot], out_hbm.at[wid, pl.ds(blk * BN, BN)],
                sem_out)

        def gather(slot):
            return pltpu.make_async_copy(
                table_sh.at[idx_bufs[slot]], out_bufs[slot], sem)

        idx_load(0, 0).start()

        off = sid * CHUNK
        for p in range(CHUNK // SUB):
            s = off + p * SUB
            pltpu.sync_copy(x_hbm.at[pl.ds(s, SUB)], bounce)
            pltpu.sync_copy(bounce, table_sh.at[pl.ds(s, SUB)])

        @pl.when(sid == 15)
        def _tail():
            pltpu.sync_copy(x_hbm.at[pl.ds(16 * CHUNK, 64)],
                            bounce.at[pl.ds(0, 64)])
            pltpu.sync_copy(bounce.at[pl.ds(0, 64)],
                            table_sh.at[pl.ds(16 * CHUNK, 64)])

        plsc.subcore_barrier()

        for blk in range(nb):
            slot = blk % 2
            if blk >= 1:
                gather(1 - slot).wait()
                out_store(blk - 1, 1 - slot).start()
            if blk + 1 < nb:
                idx_load(blk + 1, 1 - slot).start()
            idx_load(blk, slot).wait()
            if blk >= 2:
                out_store(blk - 2, slot).wait()
            gather(slot).start()

        gather((nb - 1) % 2).wait()
        out_store(nb - 1, (nb - 1) % 2).start()
        out_store(nb - 2, nb % 2).wait()
        out_store(nb - 1, (nb - 1) % 2).wait()

    return k


def kernel(x, index):
    n_out = index.shape[0] * index.shape[1]
    per_w = n_out // NW
    idx = jnp.maximum(index.astype(jnp.int32).reshape(NW, per_w), 0)
    out = _take_kernel(per_w)(x, idx)
    return jnp.minimum(out.reshape(index.shape), jnp.inf)


# speedup vs baseline: 1.3248x; 1.3248x over previous
import functools

import jax
import jax.numpy as jnp
from jax import lax
from jax.experimental import pallas as pl
from jax.experimental.pallas import tpu as pltpu
from jax.experimental.pallas import tpu_sc as plsc

NW = 32
K = 16
TABLE = 1_000_000
CHUNK = 62_496
SUB = 15_624
LANE = 128
ROW = 100
BR = 64
NB = 8


def _take_kernel(n_flat):
    bn = BR * LANE
    mesh = plsc.VectorSubcoreMesh(core_axis_name="c", subcore_axis_name="s")

    @functools.partial(
        pl.kernel,
        mesh=mesh,
        out_type=jax.ShapeDtypeStruct((n_flat,), jnp.float32),
        scratch_types=[
            pltpu.VMEM((bn,), jnp.int32),
            pltpu.VMEM((bn,), jnp.int32),
            pltpu.VMEM((bn,), jnp.float32),
            pltpu.VMEM((bn,), jnp.float32),
            pltpu.VMEM((SUB,), jnp.float32),
            pltpu.VMEM_SHARED((TABLE,), jnp.float32),
            pltpu.SemaphoreType.DMA,
            pltpu.SemaphoreType.DMA,
            pltpu.SemaphoreType.DMA,
        ],
    )
    def k(x_hbm, idx_hbm, out_hbm, idx_a, idx_b, out_a, out_b, bounce,
          table_sh, sem, sem_idx, sem_out):
        cid = lax.axis_index("c")
        sid = lax.axis_index("s")
        wid = sid * 2 + cid
        base = wid * (NB * bn)
        idx_bufs = (idx_a, idx_b)
        out_bufs = (out_a, out_b)

        def idx_load(blk, slot):
            return pltpu.make_async_copy(
                idx_hbm.at[pl.ds(base + blk * bn, bn)], idx_bufs[slot],
                sem_idx)

        def out_store(blk, slot):
            return pltpu.make_async_copy(
                out_bufs[slot], out_hbm.at[pl.ds(base + blk * bn, bn)],
                sem_out)

        idx_load(0, 0).start()

        off = sid * CHUNK
        for p in range(CHUNK // SUB):
            s = off + p * SUB
            pltpu.sync_copy(x_hbm.at[pl.ds(s, SUB)], bounce)
            pltpu.sync_copy(bounce, table_sh.at[pl.ds(s, SUB)])

        @pl.when(sid == 15)
        def _tail():
            pltpu.sync_copy(x_hbm.at[pl.ds(16 * CHUNK, 64)],
                            bounce.at[pl.ds(0, 64)])
            pltpu.sync_copy(bounce.at[pl.ds(0, 64)],
                            table_sh.at[pl.ds(16 * CHUNK, 64)])

        plsc.subcore_barrier()

        for blk in range(NB):
            slot = blk % 2
            if blk + 1 < NB:
                idx_load(blk + 1, 1 - slot).start()
            idx_load(blk, slot).wait()
            if blk >= 2:
                out_store(blk - 2, slot).wait()

            ib, ob = idx_bufs[slot], out_bufs[slot]

            def row_gather(r):
                return pltpu.make_async_copy(
                    table_sh.at[ib.at[pl.ds(r * LANE, ROW)]],
                    ob.at[pl.ds(r * LANE, ROW)], sem)

            def body(r, _):
                row_gather(r).start()

                @pl.when(r >= K)
                def _w():
                    row_gather(r - K).wait()

                return 0

            lax.fori_loop(0, BR, body, 0, unroll=False)

            def tail(j, _):
                row_gather(BR - K + j).wait()
                return 0

            lax.fori_loop(0, K, tail, 0, unroll=False)
            out_store(blk, slot).start()

        out_store(NB - 2, NB % 2).wait()
        out_store(NB - 1, 1 - NB % 2).wait()

    return k


def kernel(x, index):
    rows, row = index.shape
    pad = LANE - row
    idx_flat = jnp.pad(index.astype(jnp.int32), ((0, 0), (0, pad))).reshape(-1)
    out_flat = _take_kernel(rows * LANE)(x, idx_flat)
    return out_flat.reshape(rows, LANE)[:, :row]
